# baseline (speedup 1.0000x reference)
import jax
import jax.numpy as jnp
from jax import lax
from jax.experimental import pallas as pl
from jax.experimental.pallas import tpu as pltpu
from jax.experimental.pallas import tpu_sc as plsc

R = 128
C = 32768
CS = 4096
CT = C - CS
NC = 2
NS = 16
NW = NC * NS
RPW = R // NW
L = 16
NV = CS // L
NB = 4
KV = NV // NB
ACC = 4

TCW = 2048
NCHUNK = CT // TCW


def _shuffle(v, idx):
    dnums = lax.GatherDimensionNumbers(
        offset_dims=(), collapsed_slice_dims=(0,), start_index_map=(0,))
    return lax.gather(v, idx[:, None], dnums, (1,),
                      mode=lax.GatherScatterMode.PROMISE_IN_BOUNDS)


def _bfly(v, op):
    lane = lax.iota(jnp.int32, L)
    for s in (8, 4, 2, 1):
        v = op(v, _shuffle(v, lane ^ s))
    return v


def _scan_row(buf):
    lane = lax.iota(jnp.int32, L)
    neg = jnp.full((L,), -jnp.inf, jnp.float32)
    big = jnp.full((L,), C, jnp.int32)

    def blk(b, carry):
        g, gb = carry
        base = b * (KV * L)

        def it(i, accs):
            off = base + i * (ACC * L)
            vs = [buf[pl.ds(off + k * L, L)] for k in range(ACC)]
            return tuple(jnp.maximum(a, v) for a, v in zip(accs, vs))

        accs = lax.fori_loop(0, KV // ACC, it, (neg,) * ACC, unroll=4)
        bm = jnp.maximum(jnp.maximum(accs[0], accs[1]),
                         jnp.maximum(accs[2], accs[3]))
        gt = bm > g
        return jnp.where(gt, bm, g), jnp.where(gt, b, gb)

    g, gb = lax.fori_loop(0, NB, blk, (neg, jnp.zeros((L,), jnp.int32)))
    m = _bfly(g, jnp.maximum)

    fb = jnp.where(g == m, gb, NB)
    bstar = _bfly(fb, jnp.minimum)[0]

    ebase = bstar * (KV * L)

    def it3(i, carry):
        f0, f1, iv = carry
        v0 = buf[pl.ds(ebase + i * (2 * L), L)]
        v1 = buf[pl.ds(ebase + i * (2 * L) + L, L)]
        f0 = jnp.minimum(f0, jnp.where(v0 == m, iv, big))
        f1 = jnp.minimum(f1, jnp.where(v1 == m, iv + L, big))
        return f0, f1, iv + 2 * L

    f0, f1, _ = lax.fori_loop(0, KV // 2, it3, (big, big, lane + ebase),
                              unroll=2)
    return m, _bfly(jnp.minimum(f0, f1), jnp.minimum)


def _sc_body(x_hbm, outv_hbm, outi_hbm, buf0, buf1, resv, resi, sem0, sem1):
    cid = lax.axis_index("c")
    sid = lax.axis_index("s")
    wid = sid * NC + cid
    r0 = wid * RPW

    bufs = (buf0, buf1)
    sems = (sem0, sem1)

    cps = [pltpu.async_copy(x_hbm.at[r0 + j].at[pl.ds(0, CS)], bufs[j],
                            sems[j])
           for j in range(2)]
    for j in range(RPW):
        b = j % 2
        cps[b].wait()
        m, amax = _scan_row(bufs[b])
        if j + 2 < RPW:
            cps[b] = pltpu.async_copy(x_hbm.at[r0 + j + 2].at[pl.ds(0, CS)],
                                      bufs[b], sems[b])
        resv[j, :] = m
        resi[j, :] = amax

    pltpu.sync_copy(resv, outv_hbm.at[wid])
    pltpu.sync_copy(resi, outi_hbm.at[wid])


def _tc_body(x_ref, vout_ref, iout_ref, bestv, besti):
    step = pl.program_id(0)
    blk = x_ref[...]
    bv = jnp.max(blk, axis=1)
    ii = lax.broadcasted_iota(jnp.int32, blk.shape, 1)
    bi = jnp.min(jnp.where(blk == bv[:, None], ii, TCW), axis=1) + step * TCW

    @pl.when(step == 0)
    def _():
        bestv[...] = bv
        besti[...] = bi

    @pl.when(step > 0)
    def _():
        gt = bv > bestv[...]
        bestv[...] = jnp.where(gt, bv, bestv[...])
        besti[...] = jnp.where(gt, bi, besti[...])

    @pl.when(step == NCHUNK - 1)
    def _():
        vout_ref[...] = bestv[...]
        iout_ref[...] = besti[...]


@jax.jit
def _argmax_hybrid(x):
    mesh = plsc.VectorSubcoreMesh(core_axis_name="c", subcore_axis_name="s")
    sc_k = pl.kernel(
        _sc_body,
        mesh=mesh,
        out_type=(
            jax.ShapeDtypeStruct((NW, RPW, L), jnp.float32),
            jax.ShapeDtypeStruct((NW, RPW, L), jnp.int32),
        ),
        scratch_types=[
            pltpu.VMEM((CS,), jnp.float32),
            pltpu.VMEM((CS,), jnp.float32),
            pltpu.VMEM((RPW, L), jnp.float32),
            pltpu.VMEM((RPW, L), jnp.int32),
            pltpu.SemaphoreType.DMA,
            pltpu.SemaphoreType.DMA,
        ],
    )
    sc_v, sc_i = sc_k(x)

    tc_v, tc_i = pl.pallas_call(
        _tc_body,
        grid=(NCHUNK,),
        in_specs=[pl.BlockSpec((R, TCW), lambda i: (0, i + CS // TCW))],
        out_specs=[pl.BlockSpec((R,), lambda i: (0,)),
                   pl.BlockSpec((R,), lambda i: (0,))],
        out_shape=[jax.ShapeDtypeStruct((R,), jnp.float32),
                   jax.ShapeDtypeStruct((R,), jnp.int32)],
        scratch_shapes=[pltpu.VMEM((R,), jnp.float32),
                        pltpu.VMEM((R,), jnp.int32)],
        compiler_params=pltpu.CompilerParams(
            dimension_semantics=("arbitrary",)),
    )(x)

    scv = sc_v.reshape(R, L)[:, 0]
    sci = sc_i.reshape(R, L)[:, 0]
    return jnp.where(scv >= tc_v, sci, tc_i + CS).astype(jnp.int32)


def kernel(x):
    return _argmax_hybrid(x)


# speedup vs baseline: 1.1052x; 1.1052x over previous
import jax
import jax.numpy as jnp
from jax import lax
from jax.experimental import pallas as pl
from jax.experimental.pallas import tpu as pltpu
from jax.experimental.pallas import tpu_sc as plsc

R = 128
C = 32768
CS = 8192
CT = C - CS
NC = 2
NS = 16
NW = NC * NS
RPW = R // NW
L = 16
NV = CS // L
NB = 8
KV = NV // NB
ACC = 4

TCW = 2048
NCHUNK = CT // TCW


def _shuffle(v, idx):
    dnums = lax.GatherDimensionNumbers(
        offset_dims=(), collapsed_slice_dims=(0,), start_index_map=(0,))
    return lax.gather(v, idx[:, None], dnums, (1,),
                      mode=lax.GatherScatterMode.PROMISE_IN_BOUNDS)


def _bfly(v, op):
    lane = lax.iota(jnp.int32, L)
    for s in (8, 4, 2, 1):
        v = op(v, _shuffle(v, lane ^ s))
    return v


def _scan_row(buf):
    lane = lax.iota(jnp.int32, L)
    neg = jnp.full((L,), -jnp.inf, jnp.float32)
    big = jnp.full((L,), C, jnp.int32)

    def blk(b, carry):
        g, gb = carry
        base = b * (KV * L)

        def it(i, accs):
            off = base + i * (ACC * L)
            vs = [buf[pl.ds(off + k * L, L)] for k in range(ACC)]
            return tuple(jnp.maximum(a, v) for a, v in zip(accs, vs))

        accs = lax.fori_loop(0, KV // ACC, it, (neg,) * ACC, unroll=4)
        bm = jnp.maximum(jnp.maximum(accs[0], accs[1]),
                         jnp.maximum(accs[2], accs[3]))
        gt = bm > g
        return jnp.where(gt, bm, g), jnp.where(gt, b, gb)

    g, gb = lax.fori_loop(0, NB, blk, (neg, jnp.zeros((L,), jnp.int32)))
    m = _bfly(g, jnp.maximum)

    fb = jnp.where(g == m, gb, NB)
    bstar = _bfly(fb, jnp.minimum)[0]

    ebase = bstar * (KV * L)

    def it3(i, carry):
        f0, f1, iv = carry
        v0 = buf[pl.ds(ebase + i * (2 * L), L)]
        v1 = buf[pl.ds(ebase + i * (2 * L) + L, L)]
        f0 = jnp.minimum(f0, jnp.where(v0 == m, iv, big))
        f1 = jnp.minimum(f1, jnp.where(v1 == m, iv + L, big))
        return f0, f1, iv + 2 * L

    f0, f1, _ = lax.fori_loop(0, KV // 2, it3, (big, big, lane + ebase),
                              unroll=2)
    return m, _bfly(jnp.minimum(f0, f1), jnp.minimum)


def _sc_body(x_hbm, outv_hbm, outi_hbm, buf0, buf1, resv, resi, sem0, sem1):
    cid = lax.axis_index("c")
    sid = lax.axis_index("s")
    wid = sid * NC + cid
    r0 = wid * RPW

    bufs = (buf0, buf1)
    sems = (sem0, sem1)

    cps = [pltpu.async_copy(x_hbm.at[r0 + j].at[pl.ds(0, CS)], bufs[j],
                            sems[j])
           for j in range(2)]
    for j in range(RPW):
        b = j % 2
        cps[b].wait()
        m, amax = _scan_row(bufs[b])
        if j + 2 < RPW:
            cps[b] = pltpu.async_copy(x_hbm.at[r0 + j + 2].at[pl.ds(0, CS)],
                                      bufs[b], sems[b])
        resv[j, :] = m
        resi[j, :] = amax

    pltpu.sync_copy(resv, outv_hbm.at[wid])
    pltpu.sync_copy(resi, outi_hbm.at[wid])


def _merge_body(scv_ref, sci_ref, tcv_ref, tci_ref, out_ref):
    scv = scv_ref[...][:, 0]
    sci = sci_ref[...][:, 0]
    gt = scv >= tcv_ref[...]
    out_ref[...] = jnp.where(gt, sci, tci_ref[...] + CS)


def _tc_body(x_ref, vout_ref, iout_ref, bestv, besti):
    step = pl.program_id(0)
    blk = x_ref[...]
    bv = jnp.max(blk, axis=1)
    ii = lax.broadcasted_iota(jnp.int32, blk.shape, 1)
    bi = jnp.min(jnp.where(blk == bv[:, None], ii, TCW), axis=1) + step * TCW

    @pl.when(step == 0)
    def _():
        bestv[...] = bv
        besti[...] = bi

    @pl.when(step > 0)
    def _():
        gt = bv > bestv[...]
        bestv[...] = jnp.where(gt, bv, bestv[...])
        besti[...] = jnp.where(gt, bi, besti[...])

    @pl.when(step == NCHUNK - 1)
    def _():
        vout_ref[...] = bestv[...]
        iout_ref[...] = besti[...]


@jax.jit
def _argmax_hybrid(x):
    mesh = plsc.VectorSubcoreMesh(core_axis_name="c", subcore_axis_name="s")
    sc_k = pl.kernel(
        _sc_body,
        mesh=mesh,
        out_type=(
            jax.ShapeDtypeStruct((NW, RPW, L), jnp.float32),
            jax.ShapeDtypeStruct((NW, RPW, L), jnp.int32),
        ),
        scratch_types=[
            pltpu.VMEM((CS,), jnp.float32),
            pltpu.VMEM((CS,), jnp.float32),
            pltpu.VMEM((RPW, L), jnp.float32),
            pltpu.VMEM((RPW, L), jnp.int32),
            pltpu.SemaphoreType.DMA,
            pltpu.SemaphoreType.DMA,
        ],
    )
    sc_v, sc_i = sc_k(x)

    tc_v, tc_i = pl.pallas_call(
        _tc_body,
        grid=(NCHUNK,),
        in_specs=[pl.BlockSpec((R, TCW), lambda i: (0, i + CS // TCW))],
        out_specs=[pl.BlockSpec((R,), lambda i: (0,)),
                   pl.BlockSpec((R,), lambda i: (0,))],
        out_shape=[jax.ShapeDtypeStruct((R,), jnp.float32),
                   jax.ShapeDtypeStruct((R,), jnp.int32)],
        scratch_shapes=[pltpu.VMEM((R,), jnp.float32),
                        pltpu.VMEM((R,), jnp.int32)],
        compiler_params=pltpu.CompilerParams(
            dimension_semantics=("arbitrary",)),
    )(x)

    return pl.pallas_call(
        _merge_body,
        out_shape=jax.ShapeDtypeStruct((R,), jnp.int32),
    )(sc_v.reshape(R, L), sc_i.reshape(R, L), tc_v, tc_i)


def kernel(x):
    return _argmax_hybrid(x)
